# baseline (speedup 1.0000x reference)
import functools

import jax
import jax.numpy as jnp
from jax import lax
from jax.experimental import pallas as pl
from jax.experimental.pallas import tpu as pltpu
from jax.experimental.pallas import tpu_sc as plsc

GROUPS = 4
NE = 1024
DIM = 192
DPAD = 256
TOT = 36864
ROWS = TOT // GROUPS

TM = 2048
TMC = 1024

SPLITS = 3
TOT_S = TOT // SPLITS
ROWS_S = ROWS // SPLITS

SC_CORES = 2
SC_SUBCORES = 16
SC_WORKERS = SC_CORES * SC_SUBCORES
W_PER_G = SC_WORKERS // GROUPS
PER_W = ROWS_S // W_PER_G
CHUNK = 128
NCHUNK = PER_W // CHUNK


def _dist_argmin_body(z_ref, et_ref, ind_ref, acc_ref):
    x = z_ref[...]
    et = et_ref[...]
    en = jnp.sum(et * et, axis=0)
    s2 = lax.dot_general(
        x, et * -2.0, (((1,), (0,)), ((), ())),
        preferred_element_type=jnp.float32)
    xsq = jnp.sum(x * x, axis=1, keepdims=True)
    dist = (xsq + s2) + en[None, :]
    mind = jnp.min(dist, axis=1)
    idx = lax.broadcasted_iota(jnp.int32, (TM, NE), 1)
    ind = jnp.min(jnp.where(dist == mind[:, None], idx, NE), axis=1)

    ind_ref[...] = ind[:, None].astype(jnp.int32)

    @pl.when(pl.program_id(0) == 0)
    def _():
        acc_ref[0, 0] = 0.0

    acc_ref[0, 0] += jnp.sum(mind)


def _dist_argmin(zf, et):
    grid = (TOT_S // TM,)
    return pl.pallas_call(
        _dist_argmin_body,
        grid=grid,
        in_specs=[
            pl.BlockSpec((TM, DIM), lambda i: (i, 0)),
            pl.BlockSpec((DIM, NE), lambda i: (0, 0)),
        ],
        out_specs=[
            pl.BlockSpec((TM, 1), lambda i: (i, 0)),
            pl.BlockSpec(memory_space=pltpu.SMEM),
        ],
        out_shape=[
            jax.ShapeDtypeStruct((TOT_S, 1), jnp.int32),
            jax.ShapeDtypeStruct((1, 1), jnp.float32),
        ],
    )(zf, et)


@functools.cache
def _sc_gather_fn():
    mesh = plsc.VectorSubcoreMesh(core_axis_name="c", subcore_axis_name="s")

    @functools.partial(
        pl.kernel,
        mesh=mesh,
        out_type=jax.ShapeDtypeStruct((GROUPS, ROWS_S, DPAD), jnp.float32),
        scratch_types=[
            pltpu.VMEM((2, CHUNK), jnp.int32),
            pltpu.VMEM((CHUNK, DPAD), jnp.float32),
            pltpu.VMEM((CHUNK, DPAD), jnp.float32),
            pltpu.SemaphoreType.DMA,
            pltpu.SemaphoreType.DMA,
        ],
    )
    def _sc_gather(emb_hbm, idx_hbm, out_hbm, idx_v, rows0, rows1, sem0, sem1):
        wid = lax.axis_index("s") * SC_CORES + lax.axis_index("c")
        g = wid // W_PER_G
        mb = (wid % W_PER_G) * PER_W
        rows = (rows0, rows1)
        sems = (sem0, sem1)
        cbase = (wid % W_PER_G) * NCHUNK
        pltpu.sync_copy(idx_hbm.at[g, cbase], idx_v.at[0])
        cps = [pltpu.async_copy(emb_hbm.at[idx_v.at[0]], rows0, sem0)]
        for c in range(NCHUNK):
            b = c % 2
            if c + 1 < NCHUNK:
                nb = (c + 1) % 2
                pltpu.sync_copy(idx_hbm.at[g, cbase + c + 1], idx_v.at[nb])
                cps.append(
                    pltpu.async_copy(emb_hbm.at[idx_v.at[nb]], rows[nb],
                                     sems[nb]))
            cps[c].wait()
            pltpu.sync_copy(rows[b], out_hbm.at[g, pl.ds(mb + c * CHUNK, CHUNK)])

    return _sc_gather


def _proj_body(zq_ref, w_ref, b_ref, o_ref):
    acc = b_ref[...]
    for g in range(GROUPS):
        acc = acc + lax.dot_general(
            zq_ref[g, :, :DIM], w_ref[:, g * DIM:(g + 1) * DIM],
            (((1,), (1,)), ((), ())),
            preferred_element_type=jnp.float32)
    o_ref[...] = acc


def _project(zqp, w, b):
    c = w.shape[0]
    grid = (ROWS_S // TMC,)
    return pl.pallas_call(
        _proj_body,
        grid=grid,
        in_specs=[
            pl.BlockSpec((GROUPS, TMC, DPAD), lambda i: (0, i, 0)),
            pl.BlockSpec((c, c), lambda i: (0, 0)),
            pl.BlockSpec((1, c), lambda i: (0, 0)),
        ],
        out_specs=pl.BlockSpec((TMC, c), lambda i: (i, 0)),
        out_shape=jax.ShapeDtypeStruct((ROWS_S, c), jnp.float32),
    )(zqp, w, b)


def kernel(z, embed, W_out, b_out):
    bz, hz, cz = z.shape
    zf = z.reshape(TOT, DIM)
    et = embed.T
    embp = jnp.pad(embed, ((0, 0), (0, DPAD - DIM)))
    b2 = b_out.reshape(1, cz)
    gather = _sc_gather_fn()
    inds, outs, accs = [], [], []
    for s in range(SPLITS):
        ind2, acc = _dist_argmin(zf[s * TOT_S:(s + 1) * TOT_S], et)
        idx3 = ind2.reshape(ROWS_S, GROUPS).T.reshape(
            GROUPS, ROWS_S // CHUNK, CHUNK)
        zqp = gather(embp, idx3)
        outs.append(_project(zqp, W_out, b2))
        inds.append(ind2)
        accs.append(acc[0, 0])
    out = jnp.concatenate(outs, axis=0)
    ind_flat = jnp.concatenate(inds, axis=0).reshape(TOT)
    diff = ((accs[0] + accs[1] + accs[2]) / (TOT * DIM)) * 12.5
    return (out.reshape(bz, hz, cz), diff, ind_flat.reshape(bz, hz * GROUPS))


# speedup vs baseline: 1.2634x; 1.2634x over previous
import functools

import jax
import jax.numpy as jnp
from jax import lax
from jax.experimental import pallas as pl
from jax.experimental.pallas import tpu as pltpu
from jax.experimental.pallas import tpu_sc as plsc

GROUPS = 4
NE = 1024
DIM = 192
DPAD = 256
TOT = 36864
ROWS = TOT // GROUPS

TM = 2048
TMC = 1024

TOT_S = TOT
ROWS_S = ROWS

SC_CORES = 2
SC_SUBCORES = 16
SC_WORKERS = SC_CORES * SC_SUBCORES
W_PER_G = SC_WORKERS // GROUPS
PER_W = ROWS_S // W_PER_G
CHUNK = 128
NCHUNK = PER_W // CHUNK


def _dist_argmin_body(z_ref, et_ref, ind_ref, acc_ref):
    x = z_ref[...]
    et = et_ref[...]
    en = jnp.sum(et * et, axis=0)
    s2 = lax.dot_general(
        x, et * -2.0, (((1,), (0,)), ((), ())),
        preferred_element_type=jnp.float32)
    xsq = jnp.sum(x * x, axis=1, keepdims=True)
    dist = (xsq + s2) + en[None, :]
    mind = jnp.min(dist, axis=1)
    idx = lax.broadcasted_iota(jnp.int32, (TM, NE), 1)
    ind = jnp.min(jnp.where(dist == mind[:, None], idx, NE), axis=1)

    ind_ref[...] = ind[:, None].astype(jnp.int32)

    @pl.when(pl.program_id(0) == 0)
    def _():
        acc_ref[0, 0] = 0.0

    acc_ref[0, 0] += jnp.sum(mind)


def _dist_argmin(zf, et):
    grid = (TOT_S // TM,)
    return pl.pallas_call(
        _dist_argmin_body,
        grid=grid,
        in_specs=[
            pl.BlockSpec((TM, DIM), lambda i: (i, 0)),
            pl.BlockSpec((DIM, NE), lambda i: (0, 0)),
        ],
        out_specs=[
            pl.BlockSpec((TM, 1), lambda i: (i, 0)),
            pl.BlockSpec(memory_space=pltpu.SMEM),
        ],
        out_shape=[
            jax.ShapeDtypeStruct((TOT_S, 1), jnp.int32),
            jax.ShapeDtypeStruct((1, 1), jnp.float32),
        ],
    )(zf, et)


@functools.cache
def _sc_gather_fn():
    mesh = plsc.VectorSubcoreMesh(core_axis_name="c", subcore_axis_name="s")

    @functools.partial(
        pl.kernel,
        mesh=mesh,
        out_type=jax.ShapeDtypeStruct((GROUPS, ROWS_S, DPAD), jnp.float32),
        scratch_types=[
            pltpu.VMEM((2, CHUNK), jnp.int32),
            pltpu.VMEM((CHUNK, DPAD), jnp.float32),
            pltpu.VMEM((CHUNK, DPAD), jnp.float32),
            pltpu.SemaphoreType.DMA,
            pltpu.SemaphoreType.DMA,
        ],
    )
    def _sc_gather(emb_hbm, idx_hbm, out_hbm, idx_v, rows0, rows1, sem0, sem1):
        wid = lax.axis_index("s") * SC_CORES + lax.axis_index("c")
        g = wid // W_PER_G
        mb = (wid % W_PER_G) * PER_W
        rows = (rows0, rows1)
        sems = (sem0, sem1)
        cbase = (wid % W_PER_G) * NCHUNK
        pltpu.sync_copy(idx_hbm.at[g, cbase], idx_v.at[0])
        cps = [pltpu.async_copy(emb_hbm.at[idx_v.at[0]], rows0, sem0)]
        for c in range(NCHUNK):
            b = c % 2
            if c + 1 < NCHUNK:
                nb = (c + 1) % 2
                pltpu.sync_copy(idx_hbm.at[g, cbase + c + 1], idx_v.at[nb])
                cps.append(
                    pltpu.async_copy(emb_hbm.at[idx_v.at[nb]], rows[nb],
                                     sems[nb]))
            cps[c].wait()
            pltpu.sync_copy(rows[b], out_hbm.at[g, pl.ds(mb + c * CHUNK, CHUNK)])

    return _sc_gather


def _proj_body(zq_ref, w_ref, b_ref, o_ref):
    acc = b_ref[...]
    for g in range(GROUPS):
        acc = acc + lax.dot_general(
            zq_ref[g, :, :DIM], w_ref[:, g * DIM:(g + 1) * DIM],
            (((1,), (1,)), ((), ())),
            preferred_element_type=jnp.float32)
    o_ref[...] = acc


def _project(zqp, w, b):
    c = w.shape[0]
    grid = (ROWS_S // TMC,)
    return pl.pallas_call(
        _proj_body,
        grid=grid,
        in_specs=[
            pl.BlockSpec((GROUPS, TMC, DPAD), lambda i: (0, i, 0)),
            pl.BlockSpec((c, c), lambda i: (0, 0)),
            pl.BlockSpec((1, c), lambda i: (0, 0)),
        ],
        out_specs=pl.BlockSpec((TMC, c), lambda i: (i, 0)),
        out_shape=jax.ShapeDtypeStruct((ROWS_S, c), jnp.float32),
    )(zqp, w, b)


def kernel(z, embed, W_out, b_out):
    bz, hz, cz = z.shape
    zf = z.reshape(TOT, DIM)
    et = embed.T
    embp = jnp.pad(embed, ((0, 0), (0, DPAD - DIM)))
    b2 = b_out.reshape(1, cz)
    ind2, acc = _dist_argmin(zf, et)
    idx3 = ind2.reshape(ROWS_S, GROUPS).T.reshape(
        GROUPS, ROWS_S // CHUNK, CHUNK)
    zqp = _sc_gather_fn()(embp, idx3)
    out = _project(zqp, W_out, b2)
    ind_flat = ind2.reshape(TOT)
    diff = (acc[0, 0] / (TOT * DIM)) * 12.5
    return (out.reshape(bz, hz, cz), diff, ind_flat.reshape(bz, hz * GROUPS))


# speedup vs baseline: 1.2721x; 1.0069x over previous
import functools

import jax
import jax.numpy as jnp
from jax import lax
from jax.experimental import pallas as pl
from jax.experimental.pallas import tpu as pltpu
from jax.experimental.pallas import tpu_sc as plsc

GROUPS = 4
NE = 1024
DIM = 192
DPAD = 256
TOT = 36864
ROWS = TOT // GROUPS

TM = 3072
TMC = 1024

TOT_S = TOT
ROWS_S = ROWS

SC_CORES = 2
SC_SUBCORES = 16
SC_WORKERS = SC_CORES * SC_SUBCORES
W_PER_G = SC_WORKERS // GROUPS
PER_W = ROWS_S // W_PER_G
CHUNK = 128
NCHUNK = PER_W // CHUNK


def _dist_argmin_body(z_ref, et_ref, ind_ref, acc_ref):
    x = z_ref[...]
    et = et_ref[...]
    en = jnp.sum(et * et, axis=0)
    s2 = lax.dot_general(
        x, et * -2.0, (((1,), (0,)), ((), ())),
        preferred_element_type=jnp.float32)
    xsq = jnp.sum(x * x, axis=1, keepdims=True)
    dist = (xsq + s2) + en[None, :]
    mind = jnp.min(dist, axis=1)
    idx = lax.broadcasted_iota(jnp.int32, (TM, NE), 1)
    ind = jnp.min(jnp.where(dist == mind[:, None], idx, NE), axis=1)

    ind_ref[...] = ind[:, None].astype(jnp.int32)

    @pl.when(pl.program_id(0) == 0)
    def _():
        acc_ref[0, 0] = 0.0

    acc_ref[0, 0] += jnp.sum(mind)


def _dist_argmin(zf, et):
    grid = (TOT_S // TM,)
    return pl.pallas_call(
        _dist_argmin_body,
        grid=grid,
        in_specs=[
            pl.BlockSpec((TM, DIM), lambda i: (i, 0)),
            pl.BlockSpec((DIM, NE), lambda i: (0, 0)),
        ],
        out_specs=[
            pl.BlockSpec((TM, 1), lambda i: (i, 0)),
            pl.BlockSpec(memory_space=pltpu.SMEM),
        ],
        out_shape=[
            jax.ShapeDtypeStruct((TOT_S, 1), jnp.int32),
            jax.ShapeDtypeStruct((1, 1), jnp.float32),
        ],
    )(zf, et)


@functools.cache
def _sc_gather_fn():
    mesh = plsc.VectorSubcoreMesh(core_axis_name="c", subcore_axis_name="s")

    @functools.partial(
        pl.kernel,
        mesh=mesh,
        out_type=jax.ShapeDtypeStruct((GROUPS, ROWS_S, DPAD), jnp.float32),
        scratch_types=[
            pltpu.VMEM((2, CHUNK), jnp.int32),
            pltpu.VMEM((CHUNK, DPAD), jnp.float32),
            pltpu.VMEM((CHUNK, DPAD), jnp.float32),
            pltpu.SemaphoreType.DMA,
            pltpu.SemaphoreType.DMA,
        ],
    )
    def _sc_gather(emb_hbm, idx_hbm, out_hbm, idx_v, rows0, rows1, sem0, sem1):
        wid = lax.axis_index("s") * SC_CORES + lax.axis_index("c")
        g = wid // W_PER_G
        mb = (wid % W_PER_G) * PER_W
        rows = (rows0, rows1)
        sems = (sem0, sem1)
        cbase = (wid % W_PER_G) * NCHUNK
        pltpu.sync_copy(idx_hbm.at[g, cbase], idx_v.at[0])
        cps = [pltpu.async_copy(emb_hbm.at[idx_v.at[0]], rows0, sem0)]
        for c in range(NCHUNK):
            b = c % 2
            if c + 1 < NCHUNK:
                nb = (c + 1) % 2
                pltpu.sync_copy(idx_hbm.at[g, cbase + c + 1], idx_v.at[nb])
                cps.append(
                    pltpu.async_copy(emb_hbm.at[idx_v.at[nb]], rows[nb],
                                     sems[nb]))
            cps[c].wait()
            pltpu.sync_copy(rows[b], out_hbm.at[g, pl.ds(mb + c * CHUNK, CHUNK)])

    return _sc_gather


def _proj_body(zq_ref, w_ref, b_ref, o_ref):
    acc = b_ref[...]
    for g in range(GROUPS):
        acc = acc + lax.dot_general(
            zq_ref[g, :, :DIM], w_ref[:, g * DIM:(g + 1) * DIM],
            (((1,), (1,)), ((), ())),
            preferred_element_type=jnp.float32)
    o_ref[...] = acc


def _project(zqp, w, b):
    c = w.shape[0]
    grid = (ROWS_S // TMC,)
    return pl.pallas_call(
        _proj_body,
        grid=grid,
        in_specs=[
            pl.BlockSpec((GROUPS, TMC, DPAD), lambda i: (0, i, 0)),
            pl.BlockSpec((c, c), lambda i: (0, 0)),
            pl.BlockSpec((1, c), lambda i: (0, 0)),
        ],
        out_specs=pl.BlockSpec((TMC, c), lambda i: (i, 0)),
        out_shape=jax.ShapeDtypeStruct((ROWS_S, c), jnp.float32),
    )(zqp, w, b)


def kernel(z, embed, W_out, b_out):
    bz, hz, cz = z.shape
    zf = z.reshape(TOT, DIM)
    et = embed.T
    embp = jnp.pad(embed, ((0, 0), (0, DPAD - DIM)))
    b2 = b_out.reshape(1, cz)
    ind2, acc = _dist_argmin(zf, et)
    idx3 = ind2.reshape(ROWS_S, GROUPS).T.reshape(
        GROUPS, ROWS_S // CHUNK, CHUNK)
    zqp = _sc_gather_fn()(embp, idx3)
    out = _project(zqp, W_out, b2)
    ind_flat = ind2.reshape(TOT)
    diff = (acc[0, 0] / (TOT * DIM)) * 12.5
    return (out.reshape(bz, hz, cz), diff, ind_flat.reshape(bz, hz * GROUPS))
